# baseline (speedup 1.0000x reference)
import functools

import jax
import jax.numpy as jnp
from jax import lax
from jax.experimental import pallas as pl
from jax.experimental.pallas import tpu as pltpu
from jax.experimental.pallas import tpu_sc as plsc

_NUM_EMBEDDINGS = 8192
_DIM = 32
_TOPK = 3
_COMMITMENT = 0.25

_ROWS_PER_BLOCK = 512
_N_TOKENS = 8192


def _topk_body(x_ref, ew_ref, idx_ref, loss_ref, ewt_ref, e2_ref):
    i = pl.program_id(0)
    n = pl.num_programs(0)
    x = x_ref[...]

    @pl.when(i == 0)
    def _init():
        ewt = ew_ref[...].T
        ewt_ref[...] = ewt
        e2_ref[...] = jnp.sum(ewt * ewt, axis=0, keepdims=True)
        loss_ref[0, 0] = 0.0

    ewt = ewt_ref[...]
    x2 = jnp.sum(x * x, axis=1, keepdims=True)
    dots = jnp.dot(x * -2.0, ewt, preferred_element_type=jnp.float32)
    dist = (x2 + e2_ref[...]) + dots

    R, K = dist.shape
    big_i = jnp.int32(2**30)
    inf = jnp.float32(jnp.inf)

    v1 = dist[:, 0:128]
    v2 = jnp.full((R, 128), inf)
    v3 = jnp.full((R, 128), inf)
    b1 = jnp.zeros((R, 128), jnp.int32)
    b2 = jnp.zeros((R, 128), jnp.int32)
    b3 = jnp.zeros((R, 128), jnp.int32)
    for c in range(1, K // 128):
        d = dist[:, c * 128:(c + 1) * 128]
        bc = jnp.int32(c)
        c1 = d < v1
        c2 = d < v2
        c3 = d < v3
        v3 = jnp.where(c2, v2, jnp.where(c3, d, v3))
        b3 = jnp.where(c2, b2, jnp.where(c3, bc, b3))
        v2 = jnp.where(c1, v1, jnp.where(c2, d, v2))
        b2 = jnp.where(c1, b1, jnp.where(c2, bc, b2))
        v1 = jnp.where(c1, d, v1)
        b1 = jnp.where(c1, bc, b1)

    lane = lax.broadcasted_iota(jnp.int32, (R, 128), 1)
    g1 = b1 * 128 + lane
    g2 = b2 * 128 + lane
    g3 = b3 * 128 + lane
    total = jnp.float32(0.0)
    picked = []
    for j in range(_TOPK):
        m = jnp.min(v1, axis=1, keepdims=True)
        mi = jnp.min(jnp.where(v1 == m, g1, big_i), axis=1, keepdims=True)
        picked.append(mi)
        total += jnp.sum(m)
        if j < _TOPK - 1:
            w = g1 == mi
            v1 = jnp.where(w, v2, v1)
            g1 = jnp.where(w, g2, g1)
            v2 = jnp.where(w, v3, v2)
            g2 = jnp.where(w, g3, g2)
            v3 = jnp.where(w, inf, v3)
    idx_ref[...] = jnp.concatenate(picked, axis=1)
    loss_ref[0, 0] += total

    @pl.when(i == n - 1)
    def _finalize():
        a = loss_ref[0, 0] * jnp.float32(1.0 / (_N_TOKENS * _TOPK * _DIM))
        loss_ref[0, 0] = a + _COMMITMENT * a


def _run_topk(flat_x, ew):
    grid = _N_TOKENS // _ROWS_PER_BLOCK
    return pl.pallas_call(
        _topk_body,
        grid=(grid,),
        in_specs=[
            pl.BlockSpec((_ROWS_PER_BLOCK, _DIM), lambda i: (i, 0)),
            pl.BlockSpec((_NUM_EMBEDDINGS, _DIM), lambda i: (0, 0)),
        ],
        out_specs=[
            pl.BlockSpec((_ROWS_PER_BLOCK, _TOPK), lambda i: (i, 0)),
            pl.BlockSpec(memory_space=pltpu.SMEM),
        ],
        out_shape=[
            jax.ShapeDtypeStruct((_N_TOKENS, _TOPK), jnp.int32),
            jax.ShapeDtypeStruct((1, 1), jnp.float32),
        ],
        scratch_shapes=[
            pltpu.VMEM((_DIM, _NUM_EMBEDDINGS), jnp.float32),
            pltpu.VMEM((1, _NUM_EMBEDDINGS), jnp.float32),
        ],
    )(flat_x, ew)


_SC_WORKERS = 16
_IDX_TOTAL = _N_TOKENS * _TOPK
_IDX_PER_W = _IDX_TOTAL // _SC_WORKERS
_IDX_CHUNKS = _IDX_PER_W // 128
_BINS_PER_W = _NUM_EMBEDDINGS // _SC_WORKERS
_LANES = 16

_LOG2_C = (-3.028317481039271, 6.065830143185771, -5.2641104770847,
           3.2188328370634505, -1.2342631730389073, 0.2668588228611466,
           -0.024825606614389147)
_LN2 = 0.6931471805599453


def _sc_body(table_hbm, idx_hbm, gath_hbm, perp_hbm,
             idx_v, rows_v, ones_v, zc_v, acc_v, part_v, bins_sh, part_sh, sem):
    wid = lax.axis_index("s") * 1 + lax.axis_index("c")
    base = wid * _IDX_PER_W
    iota16 = lax.iota(jnp.int32, _LANES)
    zeros16 = jnp.zeros((_LANES,), jnp.int32)

    pltpu.sync_copy(idx_hbm.at[wid], idx_v)

    def _fill(r, _):
        def _fill_col(c, __):
            ones_v[r, pl.ds(c * _LANES, _LANES)] = jnp.ones((_LANES,), jnp.float32)
            zc_v[r, pl.ds(c * _LANES, _LANES)] = jnp.zeros((_LANES,), jnp.float32)
            return __
        return lax.fori_loop(0, 128 // _LANES, _fill_col, _)
    lax.fori_loop(0, 128, _fill, 0)

    for z in range(_BINS_PER_W // 128):
        pltpu.sync_copy(zc_v, bins_sh.at[pl.ds(wid * _BINS_PER_W + z * 128, 128)])

    for j in range(_IDX_CHUNKS):
        pltpu.async_copy(table_hbm.at[idx_v.at[j]], rows_v, sem).wait()
        pltpu.sync_copy(rows_v, gath_hbm.at[pl.ds(base + j * 128, 128)])

    plsc.subcore_barrier()
    for j in range(_IDX_CHUNKS):
        pltpu.sync_copy(ones_v, bins_sh.at[idx_v.at[j]], add=True)
    plsc.subcore_barrier()

    acc = jnp.zeros((_LANES,), jnp.float32)
    for z in range(_BINS_PER_W // 128):
        pltpu.sync_copy(bins_sh.at[pl.ds(wid * _BINS_PER_W + z * 128, 128)], zc_v)
        for g in range(128 // _LANES):
            cnt = plsc.load_gather(zc_v, [iota16 + g * _LANES, zeros16])
            p = cnt * jnp.float32(1.0 / _N_TOKENS)
            v = p + jnp.float32(1e-10)
            bits = plsc.bitcast(v, jnp.int32)
            e = (bits >> 23) - 127
            m = plsc.bitcast((bits & 0x7FFFFF) | 0x3F800000, jnp.float32)
            poly = jnp.full((_LANES,), jnp.float32(_LOG2_C[6]))
            for cf in reversed(_LOG2_C[:6]):
                poly = poly * m + jnp.float32(cf)
            l2 = poly + e.astype(jnp.float32)
            acc = acc + p * (l2 * jnp.float32(_LN2))
    acc_v[...] = acc
    pltpu.sync_copy(acc_v, part_sh.at[wid])
    plsc.subcore_barrier()

    @pl.when(wid == 0)
    def _finalize():
        pltpu.sync_copy(part_sh, part_v)
        t = jnp.zeros((_LANES,), jnp.float32)
        for r in range(_SC_WORKERS):
            t = t + part_v[r, :]
        cs = lax.cumsum(t, axis=0)
        acc_v[...] = jnp.exp(-cs)
        pltpu.sync_copy(acc_v, perp_hbm)


def _run_sc_gather(table_padded, idx3):
    mesh = plsc.VectorSubcoreMesh(core_axis_name="c", subcore_axis_name="s",
                                  num_cores=1)
    fn = functools.partial(
        pl.kernel, mesh=mesh,
        compiler_params=pltpu.CompilerParams(needs_layout_passes=False),
        out_type=[
            jax.ShapeDtypeStruct((_IDX_TOTAL, 128), jnp.float32),
            jax.ShapeDtypeStruct((_LANES,), jnp.float32),
        ],
        scratch_types=[
            pltpu.VMEM((_IDX_CHUNKS, 128), jnp.int32),
            pltpu.VMEM((128, 128), jnp.float32),
            pltpu.VMEM((128, 128), jnp.float32),
            pltpu.VMEM((128, 128), jnp.float32),
            pltpu.VMEM((_LANES,), jnp.float32),
            pltpu.VMEM((_SC_WORKERS, _LANES), jnp.float32),
            pltpu.VMEM_SHARED((_NUM_EMBEDDINGS, 128), jnp.float32),
            pltpu.VMEM_SHARED((_SC_WORKERS, _LANES), jnp.float32),
            pltpu.SemaphoreType.DMA,
        ],
    )(_sc_body)
    return fn(table_padded, idx3)


def kernel(inputs, embedding_weight):
    B, T, C = inputs.shape
    flat_x = inputs.reshape(-1, C)

    enc_idx, loss = _run_topk(flat_x, embedding_weight)

    idx3 = enc_idx.reshape(_SC_WORKERS, _IDX_CHUNKS, 128)
    table_padded = jnp.pad(embedding_weight, ((0, 0), (0, 128 - _DIM)))
    gathered, perp = _run_sc_gather(table_padded, idx3)

    quantized_st = gathered[:, :C].reshape(B, T, _TOPK, C)
    return (loss.reshape(()), quantized_st, perp[_LANES - 1].reshape(()),
            enc_idx.reshape(B, T, _TOPK))


# speedup vs baseline: 1.0013x; 1.0013x over previous
import functools

import jax
import jax.numpy as jnp
from jax import lax
from jax.experimental import pallas as pl
from jax.experimental.pallas import tpu as pltpu
from jax.experimental.pallas import tpu_sc as plsc

_NUM_EMBEDDINGS = 8192
_DIM = 32
_TOPK = 3
_COMMITMENT = 0.25

_ROWS_PER_BLOCK = 512
_N_TOKENS = 8192


def _topk_body(x_ref, ewt_ref, idx_ref, dsum_ref, e2_ref):
    i = pl.program_id(0)
    x = x_ref[...]
    ewt = ewt_ref[...]

    @pl.when(i == 0)
    def _init():
        e2_ref[...] = jnp.sum(ewt * ewt, axis=0, keepdims=True)
        dsum_ref[0, 0] = 0.0

    x2 = jnp.sum(x * x, axis=1, keepdims=True)
    dots = jnp.dot(x * -2.0, ewt, preferred_element_type=jnp.float32)
    dist = (x2 + e2_ref[...]) + dots

    R, K = dist.shape
    big_i = jnp.int32(2**30)
    inf = jnp.float32(jnp.inf)

    v1 = dist[:, 0:128]
    v2 = jnp.full((R, 128), inf)
    v3 = jnp.full((R, 128), inf)
    b1 = jnp.zeros((R, 128), jnp.int32)
    b2 = jnp.zeros((R, 128), jnp.int32)
    b3 = jnp.zeros((R, 128), jnp.int32)
    for c in range(1, K // 128):
        d = dist[:, c * 128:(c + 1) * 128]
        bc = jnp.int32(c)
        c1 = d < v1
        c2 = d < v2
        c3 = d < v3
        v3 = jnp.where(c2, v2, jnp.where(c3, d, v3))
        b3 = jnp.where(c2, b2, jnp.where(c3, bc, b3))
        v2 = jnp.where(c1, v1, jnp.where(c2, d, v2))
        b2 = jnp.where(c1, b1, jnp.where(c2, bc, b2))
        v1 = jnp.where(c1, d, v1)
        b1 = jnp.where(c1, bc, b1)

    lane = lax.broadcasted_iota(jnp.int32, (R, 128), 1)
    g1 = b1 * 128 + lane
    g2 = b2 * 128 + lane
    g3 = b3 * 128 + lane
    total = jnp.float32(0.0)
    picked = []
    for j in range(_TOPK):
        m = jnp.min(v1, axis=1, keepdims=True)
        mi = jnp.min(jnp.where(v1 == m, g1, big_i), axis=1, keepdims=True)
        picked.append(mi)
        total += jnp.sum(m)
        if j < _TOPK - 1:
            w = g1 == mi
            v1 = jnp.where(w, v2, v1)
            g1 = jnp.where(w, g2, g1)
            v2 = jnp.where(w, v3, v2)
            g2 = jnp.where(w, g3, g2)
            v3 = jnp.where(w, inf, v3)
    idx_ref[...] = jnp.concatenate(picked, axis=1)
    dsum_ref[0, 0] += total


def _run_topk(flat_x, ewt):
    grid = _N_TOKENS // _ROWS_PER_BLOCK
    return pl.pallas_call(
        _topk_body,
        grid=(grid,),
        in_specs=[
            pl.BlockSpec((_ROWS_PER_BLOCK, _DIM), lambda i: (i, 0)),
            pl.BlockSpec((_DIM, _NUM_EMBEDDINGS), lambda i: (0, 0)),
        ],
        out_specs=[
            pl.BlockSpec((_ROWS_PER_BLOCK, _TOPK), lambda i: (i, 0)),
            pl.BlockSpec(memory_space=pltpu.SMEM),
        ],
        out_shape=[
            jax.ShapeDtypeStruct((_N_TOKENS, _TOPK), jnp.int32),
            jax.ShapeDtypeStruct((1, 1), jnp.float32),
        ],
        scratch_shapes=[pltpu.VMEM((1, _NUM_EMBEDDINGS), jnp.float32)],
    )(flat_x, ewt)


_SC_WORKERS = 16
_IDX_TOTAL = _N_TOKENS * _TOPK
_IDX_PER_W = _IDX_TOTAL // _SC_WORKERS
_IDX_CHUNKS = _IDX_PER_W // 128
_BINS_PER_W = _NUM_EMBEDDINGS // _SC_WORKERS
_LANES = 16


def _sc_body(table_hbm, idx_hbm, gath_hbm, counts_hbm,
             idx_v, rows_v, ones_v, zc_v, cnt_v, bins_sh, sem):
    wid = lax.axis_index("s") * 1 + lax.axis_index("c")
    base = wid * _IDX_PER_W
    iota16 = lax.iota(jnp.int32, _LANES)
    zeros16 = jnp.zeros((_LANES,), jnp.int32)

    pltpu.sync_copy(idx_hbm.at[wid], idx_v)

    def _fill(r, _):
        def _fill_col(c, __):
            ones_v[r, pl.ds(c * _LANES, _LANES)] = jnp.ones((_LANES,), jnp.float32)
            zc_v[r, pl.ds(c * _LANES, _LANES)] = jnp.zeros((_LANES,), jnp.float32)
            return __
        return lax.fori_loop(0, 128 // _LANES, _fill_col, _)
    lax.fori_loop(0, 128, _fill, 0)

    for z in range(_BINS_PER_W // 128):
        pltpu.sync_copy(zc_v, bins_sh.at[pl.ds(wid * _BINS_PER_W + z * 128, 128)])

    for j in range(_IDX_CHUNKS):
        pltpu.async_copy(table_hbm.at[idx_v.at[j]], rows_v, sem).wait()
        pltpu.sync_copy(rows_v, gath_hbm.at[pl.ds(base + j * 128, 128)])

    plsc.subcore_barrier()
    for j in range(_IDX_CHUNKS):
        pltpu.sync_copy(ones_v, bins_sh.at[idx_v.at[j]], add=True)
    plsc.subcore_barrier()

    for z in range(_BINS_PER_W // 128):
        off = wid * _BINS_PER_W + z * 128
        pltpu.sync_copy(bins_sh.at[pl.ds(off, 128)], zc_v)
        for g in range(128 // _LANES):
            vals = plsc.load_gather(zc_v, [iota16 + g * _LANES, zeros16])
            cnt_v[pl.ds(g * _LANES, _LANES)] = vals
        pltpu.sync_copy(cnt_v, counts_hbm.at[pl.ds(off, 128)])


def _run_sc_gather(table_padded, idx3):
    mesh = plsc.VectorSubcoreMesh(core_axis_name="c", subcore_axis_name="s",
                                  num_cores=1)
    fn = functools.partial(
        pl.kernel, mesh=mesh,
        compiler_params=pltpu.CompilerParams(needs_layout_passes=False),
        out_type=[
            jax.ShapeDtypeStruct((_IDX_TOTAL, 128), jnp.float32),
            jax.ShapeDtypeStruct((_NUM_EMBEDDINGS,), jnp.float32),
        ],
        scratch_types=[
            pltpu.VMEM((_IDX_CHUNKS, 128), jnp.int32),
            pltpu.VMEM((128, 128), jnp.float32),
            pltpu.VMEM((128, 128), jnp.float32),
            pltpu.VMEM((128, 128), jnp.float32),
            pltpu.VMEM((128,), jnp.float32),
            pltpu.VMEM_SHARED((_NUM_EMBEDDINGS, 128), jnp.float32),
            pltpu.SemaphoreType.DMA,
        ],
    )(_sc_body)
    return fn(table_padded, idx3)


def _scalars_body(counts_ref, dsum_ref, loss_ref, perp_ref):
    c = counts_ref[...]
    p = c * jnp.float32(1.0 / _N_TOKENS)
    ent = jnp.sum(p * jnp.log(p + 1e-10))
    perp_ref[0, 0] = jnp.exp(-ent)
    a = dsum_ref[0, 0] * jnp.float32(1.0 / (_N_TOKENS * _TOPK * _DIM))
    loss_ref[0, 0] = a + _COMMITMENT * a


def _run_scalars(counts2d, dsum):
    return pl.pallas_call(
        _scalars_body,
        in_specs=[
            pl.BlockSpec((64, 128), lambda: (0, 0)),
            pl.BlockSpec(memory_space=pltpu.SMEM),
        ],
        out_specs=[
            pl.BlockSpec(memory_space=pltpu.SMEM),
            pl.BlockSpec(memory_space=pltpu.SMEM),
        ],
        out_shape=[
            jax.ShapeDtypeStruct((1, 1), jnp.float32),
            jax.ShapeDtypeStruct((1, 1), jnp.float32),
        ],
    )(counts2d, dsum)


def kernel(inputs, embedding_weight):
    B, T, C = inputs.shape
    flat_x = inputs.reshape(-1, C)
    ewt = embedding_weight.T

    enc_idx, dsum = _run_topk(flat_x, ewt)

    idx3 = enc_idx.reshape(_SC_WORKERS, _IDX_CHUNKS, 128)
    table_padded = jnp.pad(embedding_weight, ((0, 0), (0, 128 - _DIM)))
    gathered, counts = _run_sc_gather(table_padded, idx3)

    counts2d = counts.reshape(64, 128)
    loss, perp = _run_scalars(counts2d, dsum)

    quantized_st = gathered[:, :C].reshape(B, T, _TOPK, C)
    return (loss.reshape(()), quantized_st, perp.reshape(()),
            enc_idx.reshape(B, T, _TOPK))
